# baseline (speedup 1.0000x reference)
import functools
import math

import jax
import jax.numpy as jnp
from jax import lax
from jax.experimental import pallas as pl
from jax.experimental.pallas import tpu as pltpu
from jax.experimental.pallas import tpu_sc as plsc

E1 = math.e - 1.0
LOG_EPS = math.log(1e-10)
NS = 16
NCORES = 2
NLANES = 16
CHUNK = 128
ROW_UNROLL = 16

_C2 = (2.0 / 11.0, 2.0 / 9.0, 2.0 / 7.0, 2.0 / 5.0, 2.0 / 3.0, 2.0)


def _node_stage_body(f_ref, w_ref, b_ref, pri_ref, logp_ref, lq_ref, invq_ref):
    x = f_ref[...]
    logits = jnp.dot(x, w_ref[...], preferred_element_type=jnp.float32)
    logits = logits + b_ref[...]
    m = jnp.max(logits, axis=1, keepdims=True)
    ex = jnp.exp(logits - m)
    ssum = jnp.sum(ex, axis=1, keepdims=True)
    pri = ex / ssum
    pri_ref[...] = pri
    logp_ref[...] = jnp.maximum((logits - m) - jnp.log(ssum), LOG_EPS)
    q = E1 * pri + 1.0
    lq_ref[...] = jnp.log(q)
    invq_ref[...] = 1.0 / q


def _final_body(logp_ref, part_ref, out_ref):
    lb = logp_ref[...] + part_ref[0] + part_ref[1]
    ex = jnp.exp(lb - jnp.max(lb, axis=1, keepdims=True))
    out_ref[...] = ex / jnp.sum(ex, axis=1, keepdims=True)


def _zero_acc_slice(stage, acc, base, rpt):
    def zbody(i, carry):
        stage[i] = jnp.zeros((NLANES,), jnp.float32)
        return carry

    lax.fori_loop(0, rpt, zbody, 0)
    pltpu.sync_copy(stage, acc.at[pl.ds(base, rpt)])


def _stage_table(table_hbm, stage, table_sh, base, rpt):
    pltpu.sync_copy(table_hbm.at[pl.ds(base, rpt)], stage)
    pltpu.sync_copy(stage, table_sh.at[pl.ds(base, rpt)])


def _dump_acc_slice(stage, acc, part_hbm, c, base, rpt):
    pltpu.sync_copy(acc.at[pl.ds(base, rpt)], stage)
    pltpu.sync_copy(stage, part_hbm.at[c, pl.ds(base, rpt)])


def _sc_pass1_body(lq_hbm, sidx_hbm, didx_hbm, part_hbm,
                   sidx_v, didx_v, rows_a, rows_b, rows_c, rows_d,
                   stage, lq_sh, acc,
                   gsem_a, gsem_b, gsem_c, gsem_d,
                   ssem_a, ssem_b, ssem_c, ssem_d,
                   *, cpt, rpt):
    c = lax.axis_index("c")
    s = lax.axis_index("s")
    w = c * NS + s
    base = s * rpt

    _stage_table(lq_hbm, stage, lq_sh, base, rpt)
    _zero_acc_slice(stage, acc, base, rpt)
    pltpu.sync_copy(sidx_hbm.at[pl.ds(w * cpt, cpt)], sidx_v)
    pltpu.sync_copy(didx_hbm.at[pl.ds(w * cpt, cpt)], didx_v)
    plsc.subcore_barrier()

    def start_g(j, rows, sem):
        pltpu.async_copy(lq_sh.at[sidx_v.at[j]], rows, sem)

    def wait_g(j, rows, sem):
        pltpu.make_async_copy(lq_sh.at[sidx_v.at[j]], rows, sem).wait()

    def start_s(j, rows, sem):
        pltpu.async_copy(rows, acc.at[didx_v.at[j]], sem, add=True)

    def wait_s(j, rows, sem):
        pltpu.make_async_copy(rows, acc.at[didx_v.at[j]], sem).wait()

    bufs = (rows_a, rows_b, rows_c, rows_d)
    gsems = (gsem_a, gsem_b, gsem_c, gsem_d)
    ssems = (ssem_a, ssem_b, ssem_c, ssem_d)
    for bb in range(4):
        start_g(bb, bufs[bb], gsems[bb])

    def quad(k, carry):
        for bb in range(4):
            j = 4 * k + bb
            wait_g(j, bufs[bb], gsems[bb])
            start_s(j, bufs[bb], ssems[bb])
            jp = j - 2
            pb = (bb - 2) % 4

            @pl.when(jp >= 0)
            def _():
                wait_s(jp, bufs[pb], ssems[pb])

                @pl.when(jp + 4 < cpt)
                def _():
                    start_g(jp + 4, bufs[pb], gsems[pb])

        return carry

    lax.fori_loop(0, cpt // 4, quad, 0)
    wait_s(cpt - 2, bufs[2], ssems[2])
    wait_s(cpt - 1, bufs[3], ssems[3])
    plsc.subcore_barrier()
    _dump_acc_slice(stage, acc, part_hbm, c, base, rpt)


def _edge_rows_update(rows_y, rows_q, rows_o):
    def row_body(i, carry):
        for uu in range(ROW_UNROLL):
            r = i * ROW_UNROLL + uu
            y = rows_y[r] * rows_q[r]
            a = E1 * y
            tb = jnp.broadcast_to(jnp.sum(y), (NLANES,))
            d = a + (tb + tb)
            rec = 1.0 / d
            rec = rec * (2.0 - d * rec)
            z = a * rec
            z2 = z * z
            p = _C2[0]
            for coef in _C2[1:]:
                p = p * z2 + coef
            rows_o[r] = z * p
        return carry

    lax.fori_loop(0, CHUNK // ROW_UNROLL, row_body, 0)


def _sc_pass2_body(logp_hbm, part1_hbm, invq_hbm, sidx_hbm, didx_hbm, part_hbm,
                   sidx_v, didx_v, rows_yb, rows_qb, rows_yc, rows_qc,
                   rows_ob, rows_oc,
                   stage, pa, pb, b1_sh, invq_sh, acc,
                   sem_yb, sem_qb, sem_yc, sem_qc, sem_ob, sem_oc,
                   *, cpt, rpt):
    c = lax.axis_index("c")
    s = lax.axis_index("s")
    w = c * NS + s
    base = s * rpt

    pltpu.sync_copy(logp_hbm.at[pl.ds(base, rpt)], stage)
    pltpu.sync_copy(part1_hbm.at[0, pl.ds(base, rpt)], pa)
    pltpu.sync_copy(part1_hbm.at[1, pl.ds(base, rpt)], pb)

    def b1_body(i, carry):
        v = stage[i] + pa[i] + pb[i]
        mb = jnp.broadcast_to(jnp.max(v), (NLANES,))
        stage[i] = jnp.exp(v - mb)
        return carry

    lax.fori_loop(0, rpt, b1_body, 0)
    pltpu.sync_copy(stage, b1_sh.at[pl.ds(base, rpt)])

    _stage_table(invq_hbm, stage, invq_sh, base, rpt)
    _zero_acc_slice(stage, acc, base, rpt)
    pltpu.sync_copy(sidx_hbm.at[pl.ds(w * cpt, cpt)], sidx_v)
    pltpu.sync_copy(didx_hbm.at[pl.ds(w * cpt, cpt)], didx_v)
    plsc.subcore_barrier()

    def start_g(j, rows_y, rows_q, sem_y, sem_q):
        pltpu.async_copy(b1_sh.at[sidx_v.at[j]], rows_y, sem_y)
        pltpu.async_copy(invq_sh.at[didx_v.at[j]], rows_q, sem_q)

    def wait_g(j, rows_y, rows_q, sem_y, sem_q):
        pltpu.make_async_copy(b1_sh.at[sidx_v.at[j]], rows_y, sem_y).wait()
        pltpu.make_async_copy(invq_sh.at[didx_v.at[j]], rows_q, sem_q).wait()

    def start_s(j, ro, sem):
        pltpu.async_copy(ro, acc.at[didx_v.at[j]], sem, add=True)

    def wait_s(j, ro, sem):
        pltpu.make_async_copy(ro, acc.at[didx_v.at[j]], sem).wait()

    start_g(0, rows_yb, rows_qb, sem_yb, sem_qb)
    start_g(1, rows_yc, rows_qc, sem_yc, sem_qc)

    def pair(k, carry):
        for j, ry, rq, ro, sy, sq, so in (
            (2 * k, rows_yb, rows_qb, rows_ob, sem_yb, sem_qb, sem_ob),
            (2 * k + 1, rows_yc, rows_qc, rows_oc, sem_yc, sem_qc, sem_oc),
        ):
            wait_g(j, ry, rq, sy, sq)

            @pl.when(j >= 2)
            def _():
                wait_s(j - 2, ro, so)

            _edge_rows_update(ry, rq, ro)
            start_s(j, ro, so)

            @pl.when(j + 2 < cpt)
            def _():
                start_g(j + 2, ry, rq, sy, sq)

        return carry

    lax.fori_loop(0, cpt // 2, pair, 0)
    wait_s(cpt - 2, rows_ob, sem_ob)
    wait_s(cpt - 1, rows_oc, sem_oc)
    plsc.subcore_barrier()
    _dump_acc_slice(stage, acc, part_hbm, c, base, rpt)


def kernel(features, W, b, src_nodes, dst_nodes, rev_edges):
    N, D = features.shape
    S = W.shape[1]
    twoE = src_nodes.shape[0]
    f32 = jnp.float32

    NPAD = ((N + 1 + 127) // 128) * 128
    RPT = NPAD // NS
    NW = NCORES * NS
    CPT = -(-twoE // (CHUNK * NW))
    CPT = ((CPT + 7) // 8) * 8
    GC = NW * CPT
    EPAD = GC * CHUNK

    feats_p = jnp.zeros((NPAD, D), f32).at[:N].set(features.astype(f32))
    pad_idx = jnp.full((EPAD - twoE,), N, jnp.int32)
    sidx = jnp.concatenate([src_nodes.astype(jnp.int32), pad_idx]).reshape(GC, CHUNK)
    didx = jnp.concatenate([dst_nodes.astype(jnp.int32), pad_idx]).reshape(GC, CHUNK)
    b2 = jnp.reshape(b.astype(f32), (1, S))

    BN = NPAD // 8
    row_spec = pl.BlockSpec((BN, S), lambda i: (i, 0))
    part_spec = pl.BlockSpec((NCORES, BN, S), lambda i: (0, i, 0))
    node_sds = jax.ShapeDtypeStruct((NPAD, S), f32)

    priors, logp, lq, invq = pl.pallas_call(
        _node_stage_body,
        grid=(8,),
        in_specs=[
            pl.BlockSpec((BN, D), lambda i: (i, 0)),
            pl.BlockSpec((D, S), lambda i: (0, 0)),
            pl.BlockSpec((1, S), lambda i: (0, 0)),
        ],
        out_specs=[row_spec] * 4,
        out_shape=[node_sds] * 4,
    )(feats_p, W.astype(f32), b2)

    mesh = plsc.VectorSubcoreMesh(core_axis_name="c", subcore_axis_name="s")
    part_sds = jax.ShapeDtypeStruct((NCORES, NPAD, S), f32)
    common_scratch = [
        pltpu.VMEM((CPT, CHUNK), jnp.int32),
        pltpu.VMEM((CPT, CHUNK), jnp.int32),
    ]
    stage_scratch = pltpu.VMEM((RPT, S), f32)
    shared_table = pltpu.VMEM_SHARED((NPAD, S), f32)

    sc_params = pltpu.CompilerParams(
        use_tc_tiling_on_sc=False, needs_layout_passes=False
    )
    pass1 = pl.kernel(
        functools.partial(_sc_pass1_body, cpt=CPT, rpt=RPT),
        out_type=part_sds,
        mesh=mesh,
        compiler_params=sc_params,
        scratch_types=common_scratch
        + [pltpu.VMEM((CHUNK, S), f32)] * 4
        + [stage_scratch, shared_table, shared_table]
        + [pltpu.SemaphoreType.DMA] * 8,
    )
    part1 = pass1(lq, sidx, didx)

    pass2 = pl.kernel(
        functools.partial(_sc_pass2_body, cpt=CPT, rpt=RPT),
        out_type=part_sds,
        mesh=mesh,
        compiler_params=sc_params,
        scratch_types=common_scratch
        + [pltpu.VMEM((CHUNK, S), f32)] * 6
        + [stage_scratch, stage_scratch, stage_scratch]
        + [shared_table, shared_table, shared_table]
        + [pltpu.SemaphoreType.DMA] * 6,
    )
    part2 = pass2(logp, part1, invq, sidx, didx)

    beliefs = pl.pallas_call(
        _final_body,
        grid=(8,),
        in_specs=[row_spec, part_spec],
        out_specs=row_spec,
        out_shape=node_sds,
    )(logp, part2)

    return (priors[:N], beliefs[:N])


# speedup vs baseline: 1.0314x; 1.0314x over previous
import functools
import math

import jax
import jax.numpy as jnp
from jax import lax
from jax.experimental import pallas as pl
from jax.experimental.pallas import tpu as pltpu
from jax.experimental.pallas import tpu_sc as plsc

E1 = math.e - 1.0
LOG_EPS = math.log(1e-10)
NS = 16
NCORES = 2
NLANES = 16
CHUNK = 256
ROW_UNROLL = 8

_C2 = (2.0 / 11.0, 2.0 / 9.0, 2.0 / 7.0, 2.0 / 5.0, 2.0 / 3.0, 2.0)


def _node_stage_body(f_ref, w_ref, b_ref, pri_ref, logp_ref, lq_ref, invq_ref):
    x = f_ref[...]
    logits = jnp.dot(x, w_ref[...], preferred_element_type=jnp.float32)
    logits = logits + b_ref[...]
    m = jnp.max(logits, axis=1, keepdims=True)
    ex = jnp.exp(logits - m)
    ssum = jnp.sum(ex, axis=1, keepdims=True)
    pri = ex / ssum
    pri_ref[...] = pri
    logp_ref[...] = jnp.maximum((logits - m) - jnp.log(ssum), LOG_EPS)
    q = E1 * pri + 1.0
    lq_ref[...] = jnp.log(q)
    invq_ref[...] = 1.0 / q


def _final_body(logp_ref, part_ref, out_ref):
    lb = logp_ref[...] + part_ref[0] + part_ref[1]
    ex = jnp.exp(lb - jnp.max(lb, axis=1, keepdims=True))
    out_ref[...] = ex / jnp.sum(ex, axis=1, keepdims=True)


def _zero_acc_slice(stage, acc, base, rpt):
    def zbody(i, carry):
        stage[i] = jnp.zeros((NLANES,), jnp.float32)
        return carry

    lax.fori_loop(0, rpt, zbody, 0)
    pltpu.sync_copy(stage, acc.at[pl.ds(base, rpt)])


def _stage_table(table_hbm, stage, table_sh, base, rpt):
    pltpu.sync_copy(table_hbm.at[pl.ds(base, rpt)], stage)
    pltpu.sync_copy(stage, table_sh.at[pl.ds(base, rpt)])


def _dump_acc_slice(stage, acc, part_hbm, c, base, rpt):
    pltpu.sync_copy(acc.at[pl.ds(base, rpt)], stage)
    pltpu.sync_copy(stage, part_hbm.at[c, pl.ds(base, rpt)])


def _sc_pass1_body(lq_hbm, sidx_hbm, didx_hbm, part_hbm,
                   sidx_v, didx_v, rows_a, rows_b, rows_c, rows_d,
                   stage, lq_sh, acc,
                   gsem_a, gsem_b, gsem_c, gsem_d,
                   ssem_a, ssem_b, ssem_c, ssem_d,
                   *, cpt, rpt):
    c = lax.axis_index("c")
    s = lax.axis_index("s")
    w = c * NS + s
    base = s * rpt

    _stage_table(lq_hbm, stage, lq_sh, base, rpt)
    _zero_acc_slice(stage, acc, base, rpt)
    pltpu.sync_copy(sidx_hbm.at[pl.ds(w * cpt, cpt)], sidx_v)
    pltpu.sync_copy(didx_hbm.at[pl.ds(w * cpt, cpt)], didx_v)
    plsc.subcore_barrier()

    def start_g(j, rows, sem):
        pltpu.async_copy(lq_sh.at[sidx_v.at[j]], rows, sem)

    def wait_g(j, rows, sem):
        pltpu.make_async_copy(lq_sh.at[sidx_v.at[j]], rows, sem).wait()

    def start_s(j, rows, sem):
        pltpu.async_copy(rows, acc.at[didx_v.at[j]], sem, add=True)

    def wait_s(j, rows, sem):
        pltpu.make_async_copy(rows, acc.at[didx_v.at[j]], sem).wait()

    bufs = (rows_a, rows_b, rows_c, rows_d)
    gsems = (gsem_a, gsem_b, gsem_c, gsem_d)
    ssems = (ssem_a, ssem_b, ssem_c, ssem_d)
    for bb in range(4):
        start_g(bb, bufs[bb], gsems[bb])

    def quad(k, carry):
        for bb in range(4):
            j = 4 * k + bb
            wait_g(j, bufs[bb], gsems[bb])
            start_s(j, bufs[bb], ssems[bb])
            jp = j - 2
            pb = (bb - 2) % 4

            @pl.when(jp >= 0)
            def _():
                wait_s(jp, bufs[pb], ssems[pb])

                @pl.when(jp + 4 < cpt)
                def _():
                    start_g(jp + 4, bufs[pb], gsems[pb])

        return carry

    lax.fori_loop(0, cpt // 4, quad, 0)
    wait_s(cpt - 2, bufs[2], ssems[2])
    wait_s(cpt - 1, bufs[3], ssems[3])
    plsc.subcore_barrier()
    _dump_acc_slice(stage, acc, part_hbm, c, base, rpt)


def _edge_rows_update(rows_y, rows_q, rows_o):
    def row_body(i, carry):
        for uu in range(ROW_UNROLL):
            r = i * ROW_UNROLL + uu
            y = rows_y[r] * rows_q[r]
            a = E1 * y
            tb = jnp.broadcast_to(jnp.sum(y), (NLANES,))
            d = a + (tb + tb)
            rec = 1.0 / d
            rec = rec * (2.0 - d * rec)
            z = a * rec
            z2 = z * z
            p = _C2[0]
            for coef in _C2[1:]:
                p = p * z2 + coef
            rows_o[r] = z * p
        return carry

    lax.fori_loop(0, CHUNK // ROW_UNROLL, row_body, 0)


def _sc_pass2_body(logp_hbm, part1_hbm, invq_hbm, sidx_hbm, didx_hbm, part_hbm,
                   sidx_v, didx_v, rows_yb, rows_qb, rows_yc, rows_qc,
                   rows_ob, rows_oc,
                   stage, pa, pb, b1_sh, invq_sh, acc,
                   sem_yb, sem_qb, sem_yc, sem_qc, sem_ob, sem_oc,
                   *, cpt, rpt):
    c = lax.axis_index("c")
    s = lax.axis_index("s")
    w = c * NS + s
    base = s * rpt

    pltpu.sync_copy(logp_hbm.at[pl.ds(base, rpt)], stage)
    pltpu.sync_copy(part1_hbm.at[0, pl.ds(base, rpt)], pa)
    pltpu.sync_copy(part1_hbm.at[1, pl.ds(base, rpt)], pb)

    def b1_body(i, carry):
        v = stage[i] + pa[i] + pb[i]
        mb = jnp.broadcast_to(jnp.max(v), (NLANES,))
        stage[i] = jnp.exp(v - mb)
        return carry

    lax.fori_loop(0, rpt, b1_body, 0)
    pltpu.sync_copy(stage, b1_sh.at[pl.ds(base, rpt)])

    _stage_table(invq_hbm, stage, invq_sh, base, rpt)
    _zero_acc_slice(stage, acc, base, rpt)
    pltpu.sync_copy(sidx_hbm.at[pl.ds(w * cpt, cpt)], sidx_v)
    pltpu.sync_copy(didx_hbm.at[pl.ds(w * cpt, cpt)], didx_v)
    plsc.subcore_barrier()

    def start_g(j, rows_y, rows_q, sem_y, sem_q):
        pltpu.async_copy(b1_sh.at[sidx_v.at[j]], rows_y, sem_y)
        pltpu.async_copy(invq_sh.at[didx_v.at[j]], rows_q, sem_q)

    def wait_g(j, rows_y, rows_q, sem_y, sem_q):
        pltpu.make_async_copy(b1_sh.at[sidx_v.at[j]], rows_y, sem_y).wait()
        pltpu.make_async_copy(invq_sh.at[didx_v.at[j]], rows_q, sem_q).wait()

    def start_s(j, ro, sem):
        pltpu.async_copy(ro, acc.at[didx_v.at[j]], sem, add=True)

    def wait_s(j, ro, sem):
        pltpu.make_async_copy(ro, acc.at[didx_v.at[j]], sem).wait()

    start_g(0, rows_yb, rows_qb, sem_yb, sem_qb)
    start_g(1, rows_yc, rows_qc, sem_yc, sem_qc)

    def pair(k, carry):
        for j, ry, rq, ro, sy, sq, so in (
            (2 * k, rows_yb, rows_qb, rows_ob, sem_yb, sem_qb, sem_ob),
            (2 * k + 1, rows_yc, rows_qc, rows_oc, sem_yc, sem_qc, sem_oc),
        ):
            wait_g(j, ry, rq, sy, sq)

            @pl.when(j >= 2)
            def _():
                wait_s(j - 2, ro, so)

            _edge_rows_update(ry, rq, ro)
            start_s(j, ro, so)

            @pl.when(j + 2 < cpt)
            def _():
                start_g(j + 2, ry, rq, sy, sq)

        return carry

    lax.fori_loop(0, cpt // 2, pair, 0)
    wait_s(cpt - 2, rows_ob, sem_ob)
    wait_s(cpt - 1, rows_oc, sem_oc)
    plsc.subcore_barrier()
    _dump_acc_slice(stage, acc, part_hbm, c, base, rpt)


def kernel(features, W, b, src_nodes, dst_nodes, rev_edges):
    N, D = features.shape
    S = W.shape[1]
    twoE = src_nodes.shape[0]
    f32 = jnp.float32

    NPAD = ((N + 1 + 127) // 128) * 128
    RPT = NPAD // NS
    NW = NCORES * NS
    CPT = -(-twoE // (CHUNK * NW))
    CPT = ((CPT + 7) // 8) * 8
    GC = NW * CPT
    EPAD = GC * CHUNK

    feats_p = jnp.zeros((NPAD, D), f32).at[:N].set(features.astype(f32))
    pad_idx = jnp.full((EPAD - twoE,), N, jnp.int32)
    sidx = jnp.concatenate([src_nodes.astype(jnp.int32), pad_idx]).reshape(GC, CHUNK)
    didx = jnp.concatenate([dst_nodes.astype(jnp.int32), pad_idx]).reshape(GC, CHUNK)
    b2 = jnp.reshape(b.astype(f32), (1, S))

    BN = NPAD // 8
    row_spec = pl.BlockSpec((BN, S), lambda i: (i, 0))
    part_spec = pl.BlockSpec((NCORES, BN, S), lambda i: (0, i, 0))
    node_sds = jax.ShapeDtypeStruct((NPAD, S), f32)

    priors, logp, lq, invq = pl.pallas_call(
        _node_stage_body,
        grid=(8,),
        in_specs=[
            pl.BlockSpec((BN, D), lambda i: (i, 0)),
            pl.BlockSpec((D, S), lambda i: (0, 0)),
            pl.BlockSpec((1, S), lambda i: (0, 0)),
        ],
        out_specs=[row_spec] * 4,
        out_shape=[node_sds] * 4,
    )(feats_p, W.astype(f32), b2)

    mesh = plsc.VectorSubcoreMesh(core_axis_name="c", subcore_axis_name="s")
    part_sds = jax.ShapeDtypeStruct((NCORES, NPAD, S), f32)
    common_scratch = [
        pltpu.VMEM((CPT, CHUNK), jnp.int32),
        pltpu.VMEM((CPT, CHUNK), jnp.int32),
    ]
    stage_scratch = pltpu.VMEM((RPT, S), f32)
    shared_table = pltpu.VMEM_SHARED((NPAD, S), f32)

    sc_params = pltpu.CompilerParams(
        use_tc_tiling_on_sc=False, needs_layout_passes=False
    )
    pass1 = pl.kernel(
        functools.partial(_sc_pass1_body, cpt=CPT, rpt=RPT),
        out_type=part_sds,
        mesh=mesh,
        compiler_params=sc_params,
        scratch_types=common_scratch
        + [pltpu.VMEM((CHUNK, S), f32)] * 4
        + [stage_scratch, shared_table, shared_table]
        + [pltpu.SemaphoreType.DMA] * 8,
    )
    part1 = pass1(lq, sidx, didx)

    pass2 = pl.kernel(
        functools.partial(_sc_pass2_body, cpt=CPT, rpt=RPT),
        out_type=part_sds,
        mesh=mesh,
        compiler_params=sc_params,
        scratch_types=common_scratch
        + [pltpu.VMEM((CHUNK, S), f32)] * 6
        + [stage_scratch, stage_scratch, stage_scratch]
        + [shared_table, shared_table, shared_table]
        + [pltpu.SemaphoreType.DMA] * 6,
    )
    part2 = pass2(logp, part1, invq, sidx, didx)

    beliefs = pl.pallas_call(
        _final_body,
        grid=(8,),
        in_specs=[row_spec, part_spec],
        out_specs=row_spec,
        out_shape=node_sds,
    )(logp, part2)

    return (priors[:N], beliefs[:N])
